# baseline (speedup 1.0000x reference)
import dataclasses
import functools

import jax
import jax.numpy as jnp
from jax import lax
from jax.experimental import pallas as pl
from jax.experimental.pallas import tpu as pltpu
from jax.experimental.pallas import tpu_sc as plsc

DIM = 32
N_CHUNKS = 4
NUM_CORES = 2
NUM_SUBCORES = 16
NUM_WORKERS = NUM_CORES * NUM_SUBCORES
LANES = 16

VOCAB = 1000000
V1 = (VOCAB // 2 // 128) * 128
V2 = VOCAB - V1
OFF2 = N_CHUNKS * V1

BATCH = 16384
BPW = BATCH // NUM_WORKERS
TOK_CHUNK = 128
N_TOK_CHUNKS = BPW // TOK_CHUNK


def _sc_body(x_hbm, table_hbm, wa_hbm, wb_hbm, ha_hbm, hb_hbm, out_hbm,
             x_v, idx_v, idxb_v, hcol_v, wcol_v, hcolb_v, wcolb_v,
             vecs_v, out_v,
             sem_x, sem_hw, sem_tab, sem_out):
    wid = lax.axis_index("s") * NUM_CORES + lax.axis_index("c")
    base = wid * BPW
    x_cps = [
        pltpu.async_copy(
            x_hbm.at[pl.ds(base + j * TOK_CHUNK, TOK_CHUNK)],
            x_v.at[j], sem_x)
        for j in range(N_TOK_CHUNKS)
    ]
    for cp in x_cps:
        cp.wait()

    for j in range(N_TOK_CHUNKS):
        @pl.loop(0, TOK_CHUNK // LANES)
        def _(t, j=j):
            s = pl.ds(t * LANES, LANES)
            xv = x_v[j, s]
            xa = jnp.minimum(xv, V1 - 1)
            xb = jnp.clip(xv - V1, 0, V2 - 1)
            for c in range(N_CHUNKS):
                idx_v[j * N_CHUNKS + c, s] = xa + (c * V1)
                idxb_v[j * N_CHUNKS + c, s] = xb + (c * V2)

    def fire_hw(j):
        cps = []
        for c in range(N_CHUNKS):
            r = j * N_CHUNKS + c
            cps.append(pltpu.async_copy(
                ha_hbm.at[idx_v.at[r]], hcol_v.at[r], sem_hw.at[j]))
            cps.append(pltpu.async_copy(
                wa_hbm.at[idx_v.at[r]], wcol_v.at[r], sem_hw.at[j]))
            cps.append(pltpu.async_copy(
                hb_hbm.at[idxb_v.at[r]], hcolb_v.at[r], sem_hw.at[j]))
            cps.append(pltpu.async_copy(
                wb_hbm.at[idxb_v.at[r]], wcolb_v.at[r], sem_hw.at[j]))
        return cps

    def select_half(j):
        @pl.loop(0, TOK_CHUNK // LANES)
        def _(t, j=j):
            s = pl.ds(t * LANES, LANES)
            m = x_v[j, s] < V1
            for c in range(N_CHUNKS):
                r = j * N_CHUNKS + c
                hcol_v[r, s] = jnp.where(m, hcol_v[r, s], hcolb_v[r, s])
                wcol_v[r, s] = jnp.where(m, wcol_v[r, s], wcolb_v[r, s])

    def fire_tab(j):
        p = j % 2
        cps = []
        for c in range(N_CHUNKS):
            cps.append(pltpu.async_copy(
                table_hbm.at[hcol_v.at[j * N_CHUNKS + c]],
                vecs_v.at[pl.ds((p * N_CHUNKS + c) * TOK_CHUNK, TOK_CHUNK)],
                sem_tab.at[p]))
        return cps

    def compute(j):
        p = j % 2
        row0 = p * N_CHUNKS * TOK_CHUNK
        wrow0 = j * N_CHUNKS

        @pl.loop(0, TOK_CHUNK)
        def _(b):
            brow = jnp.full((LANES,), b, jnp.int32)
            wv = plsc.load_gather(
                wcol_v, [jnp.full((LANES,), wrow0, jnp.int32), brow])
            acc_lo = wv * vecs_v[row0 + b, pl.ds(0, LANES)]
            acc_hi = wv * vecs_v[row0 + b, pl.ds(LANES, LANES)]
            for c in range(1, N_CHUNKS):
                wv = plsc.load_gather(
                    wcol_v, [jnp.full((LANES,), wrow0 + c, jnp.int32), brow])
                r = row0 + c * TOK_CHUNK + b
                acc_lo = acc_lo + wv * vecs_v[r, pl.ds(0, LANES)]
                acc_hi = acc_hi + wv * vecs_v[r, pl.ds(LANES, LANES)]
            out_v[p * TOK_CHUNK + b, pl.ds(0, LANES)] = acc_lo
            out_v[p * TOK_CHUNK + b, pl.ds(LANES, LANES)] = acc_hi

    def fire_out(j):
        p = j % 2
        return pltpu.async_copy(
            out_v.at[pl.ds(p * TOK_CHUNK, TOK_CHUNK)],
            out_hbm.at[pl.ds(base + j * TOK_CHUNK, TOK_CHUNK)],
            sem_out.at[p])

    hw_cps = {0: fire_hw(0), 1: fire_hw(1)}
    for cp in hw_cps[0]:
        cp.wait()
    select_half(0)
    tab_cps = {0: fire_tab(0)}
    out_cps = {}
    for j in range(N_TOK_CHUNKS):
        if j + 2 < N_TOK_CHUNKS:
            hw_cps[j + 2] = fire_hw(j + 2)
        for cp in tab_cps[j]:
            cp.wait()
        if j + 1 < N_TOK_CHUNKS:
            for cp in hw_cps[j + 1]:
                cp.wait()
            select_half(j + 1)
            tab_cps[j + 1] = fire_tab(j + 1)
        if j - 2 >= 0:
            out_cps[j - 2].wait()
        compute(j)
        out_cps[j] = fire_out(j)
    out_cps[N_TOK_CHUNKS - 2].wait()
    out_cps[N_TOK_CHUNKS - 1].wait()


@functools.lru_cache(maxsize=1)
def _build_kernel():
    mesh = plsc.VectorSubcoreMesh(core_axis_name="c", subcore_axis_name="s")
    cp = pltpu.CompilerParams()
    fields = pltpu.CompilerParams.__dataclass_fields__
    if "needs_layout_passes" in fields:
        cp = dataclasses.replace(cp, needs_layout_passes=False)
    if "use_tc_tiling_on_sc" in fields:
        cp = dataclasses.replace(cp, use_tc_tiling_on_sc=False)
    n_idx = N_TOK_CHUNKS * N_CHUNKS
    return pl.kernel(
        _sc_body,
        out_type=jax.ShapeDtypeStruct((BATCH, DIM), jnp.float32),
        mesh=mesh,
        compiler_params=cp,
        scratch_types=[
            pltpu.VMEM((N_TOK_CHUNKS, TOK_CHUNK), jnp.int32),
            pltpu.VMEM((n_idx, TOK_CHUNK), jnp.int32),
            pltpu.VMEM((n_idx, TOK_CHUNK), jnp.int32),
            pltpu.VMEM((n_idx, TOK_CHUNK), jnp.int32),
            pltpu.VMEM((n_idx, TOK_CHUNK), jnp.float32),
            pltpu.VMEM((n_idx, TOK_CHUNK), jnp.int32),
            pltpu.VMEM((n_idx, TOK_CHUNK), jnp.float32),
            pltpu.VMEM((2 * N_CHUNKS * TOK_CHUNK, DIM), jnp.float32),
            pltpu.VMEM((2 * TOK_CHUNK, DIM), jnp.float32),
            pltpu.SemaphoreType.DMA,
            pltpu.SemaphoreType.DMA((N_TOK_CHUNKS,)),
            pltpu.SemaphoreType.DMA((2,)),
            pltpu.SemaphoreType.DMA((2,)),
        ],
    )


def kernel(x, table, weights, h):
    x = x.astype(jnp.int32)
    h = h.astype(jnp.int32)
    ha = h[:V1].T.reshape(-1)
    hb = h[V1:].T.reshape(-1)
    wa = weights[:V1].T.reshape(-1)
    wb = weights[V1:].T.reshape(-1)
    return _build_kernel()(x, table, wa, wb, ha, hb)


# speedup vs baseline: 1.4498x; 1.4498x over previous
import dataclasses
import functools

import jax
import jax.numpy as jnp
from jax import lax
from jax.experimental import pallas as pl
from jax.experimental.pallas import tpu as pltpu
from jax.experimental.pallas import tpu_sc as plsc

DIM = 32
N_CHUNKS = 4
NUM_CORES = 2
NUM_SUBCORES = 16
NUM_WORKERS = NUM_CORES * NUM_SUBCORES
LANES = 16

VOCAB = 1000000

BATCH = 16384
BPW = BATCH // NUM_WORKERS
TOK_CHUNK = 128
N_TOK_CHUNKS = BPW // TOK_CHUNK


def _sc_body(x_hbm, table_hbm, w_hbm, h_hbm, out_hbm,
             x_v, idx_v, hcol_v, wcol_v, vecs_v, out_v,
             sem_x, sem_hw, sem_tab, sem_out):
    wid = lax.axis_index("s") * NUM_CORES + lax.axis_index("c")
    base = wid * BPW
    x_cps = [
        pltpu.async_copy(
            x_hbm.at[pl.ds(base + j * TOK_CHUNK, TOK_CHUNK)],
            x_v.at[j], sem_x)
        for j in range(N_TOK_CHUNKS)
    ]
    for cp in x_cps:
        cp.wait()

    for j in range(N_TOK_CHUNKS):
        @pl.loop(0, TOK_CHUNK // LANES)
        def _(t, j=j):
            s = pl.ds(t * LANES, LANES)
            xv = x_v[j, s]
            for c in range(N_CHUNKS):
                idx_v[j * N_CHUNKS + c, s] = xv

    def fire_hw(j):
        cps = []
        for c in range(N_CHUNKS):
            r = j * N_CHUNKS + c
            cps.append(pltpu.async_copy(
                h_hbm.at[c].at[idx_v.at[r]], hcol_v.at[r], sem_hw.at[j]))
            cps.append(pltpu.async_copy(
                w_hbm.at[c].at[idx_v.at[r]], wcol_v.at[r], sem_hw.at[j]))
        return cps

    def fire_tab(j):
        p = j % 2
        cps = []
        for c in range(N_CHUNKS):
            cps.append(pltpu.async_copy(
                table_hbm.at[hcol_v.at[j * N_CHUNKS + c]],
                vecs_v.at[pl.ds((p * N_CHUNKS + c) * TOK_CHUNK, TOK_CHUNK)],
                sem_tab.at[p]))
        return cps

    def compute(j):
        p = j % 2
        row0 = p * N_CHUNKS * TOK_CHUNK
        wrow0 = j * N_CHUNKS

        @pl.loop(0, TOK_CHUNK)
        def _(b):
            brow = jnp.full((LANES,), b, jnp.int32)
            wv = plsc.load_gather(
                wcol_v, [jnp.full((LANES,), wrow0, jnp.int32), brow])
            acc_lo = wv * vecs_v[row0 + b, pl.ds(0, LANES)]
            acc_hi = wv * vecs_v[row0 + b, pl.ds(LANES, LANES)]
            for c in range(1, N_CHUNKS):
                wv = plsc.load_gather(
                    wcol_v, [jnp.full((LANES,), wrow0 + c, jnp.int32), brow])
                r = row0 + c * TOK_CHUNK + b
                acc_lo = acc_lo + wv * vecs_v[r, pl.ds(0, LANES)]
                acc_hi = acc_hi + wv * vecs_v[r, pl.ds(LANES, LANES)]
            out_v[p * TOK_CHUNK + b, pl.ds(0, LANES)] = acc_lo
            out_v[p * TOK_CHUNK + b, pl.ds(LANES, LANES)] = acc_hi

    def fire_out(j):
        p = j % 2
        return pltpu.async_copy(
            out_v.at[pl.ds(p * TOK_CHUNK, TOK_CHUNK)],
            out_hbm.at[pl.ds(base + j * TOK_CHUNK, TOK_CHUNK)],
            sem_out.at[p])

    hw_cps = {0: fire_hw(0), 1: fire_hw(1)}
    for cp in hw_cps[0]:
        cp.wait()
    tab_cps = {0: fire_tab(0)}
    out_cps = {}
    for j in range(N_TOK_CHUNKS):
        if j + 2 < N_TOK_CHUNKS:
            hw_cps[j + 2] = fire_hw(j + 2)
        for cp in tab_cps[j]:
            cp.wait()
        if j + 1 < N_TOK_CHUNKS:
            for cp in hw_cps[j + 1]:
                cp.wait()
            tab_cps[j + 1] = fire_tab(j + 1)
        if j - 2 >= 0:
            out_cps[j - 2].wait()
        compute(j)
        out_cps[j] = fire_out(j)
    out_cps[N_TOK_CHUNKS - 2].wait()
    out_cps[N_TOK_CHUNKS - 1].wait()


@functools.lru_cache(maxsize=1)
def _build_kernel():
    mesh = plsc.VectorSubcoreMesh(core_axis_name="c", subcore_axis_name="s")
    cp = pltpu.CompilerParams()
    fields = pltpu.CompilerParams.__dataclass_fields__
    if "needs_layout_passes" in fields:
        cp = dataclasses.replace(cp, needs_layout_passes=False)
    if "use_tc_tiling_on_sc" in fields:
        cp = dataclasses.replace(cp, use_tc_tiling_on_sc=False)
    n_idx = N_TOK_CHUNKS * N_CHUNKS
    return pl.kernel(
        _sc_body,
        out_type=jax.ShapeDtypeStruct((BATCH, DIM), jnp.float32),
        mesh=mesh,
        compiler_params=cp,
        scratch_types=[
            pltpu.VMEM((N_TOK_CHUNKS, TOK_CHUNK), jnp.int32),
            pltpu.VMEM((n_idx, TOK_CHUNK), jnp.int32),
            pltpu.VMEM((n_idx, TOK_CHUNK), jnp.int32),
            pltpu.VMEM((n_idx, TOK_CHUNK), jnp.float32),
            pltpu.VMEM((2 * N_CHUNKS * TOK_CHUNK, DIM), jnp.float32),
            pltpu.VMEM((2 * TOK_CHUNK, DIM), jnp.float32),
            pltpu.SemaphoreType.DMA,
            pltpu.SemaphoreType.DMA((N_TOK_CHUNKS,)),
            pltpu.SemaphoreType.DMA((2,)),
            pltpu.SemaphoreType.DMA((2,)),
        ],
    )


def kernel(x, table, weights, h):
    x = x.astype(jnp.int32)
    hflat = jnp.swapaxes(h.astype(jnp.int32), 0, 1)
    wflat = jnp.swapaxes(weights, 0, 1)
    return _build_kernel()(x, table, wflat, hflat)
